# baseline (speedup 1.0000x reference)
import math

import jax
import jax.numpy as jnp
from jax import lax
from jax.experimental import pallas as pl
from jax.experimental.pallas import tpu as pltpu

_B, _T, _D, _H = 8, 64, 1024, 16
_HD = _D // _H
_BT = _B * _T
_SCALE = 1.0 / math.sqrt(_HD)
_RB = 256


def _fused_attn_kernel(x_ref, wqkv_ref, wproj_ref, o_ref, y_ref):
    x = x_ref[...]
    qkv = jnp.dot(x, wqkv_ref[...], preferred_element_type=jnp.float32)
    q = qkv[:, 0 * _D:1 * _D] * _SCALE
    k = qkv[:, 1 * _D:2 * _D]
    v = qkv[:, 2 * _D:3 * _D]

    row = lax.broadcasted_iota(jnp.int32, (_RB, _RB), 0)
    col = lax.broadcasted_iota(jnp.int32, (_RB, _RB), 1)
    keep = jnp.logical_and((row // _T) == (col // _T), col <= row)
    bias = jnp.where(keep, 0.0, -1e30)

    for h in range(_H):
        c0 = h * _HD
        qh = q[:, c0:c0 + _HD]
        kh = k[:, c0:c0 + _HD]
        vh = v[:, c0:c0 + _HD]
        s = lax.dot_general(qh, kh, (((1,), (1,)), ((), ())),
                            preferred_element_type=jnp.float32) + bias
        s = s - jnp.max(s, axis=-1, keepdims=True)
        p = jnp.exp(s)
        p = p / jnp.sum(p, axis=-1, keepdims=True)
        y_ref[:, c0:c0 + _HD] = jnp.dot(p, vh,
                                        preferred_element_type=jnp.float32)

    o_ref[...] = jnp.dot(y_ref[...], wproj_ref[...],
                         preferred_element_type=jnp.float32)


@jax.jit
def kernel(x, w_qkv, w_proj):
    x2d = x.reshape(_BT, _D)
    y2d = pl.pallas_call(
        _fused_attn_kernel,
        out_shape=jax.ShapeDtypeStruct((_BT, _D), jnp.float32),
        grid=(_BT // _RB,),
        in_specs=[
            pl.BlockSpec((_RB, _D), lambda i: (i, 0)),
            pl.BlockSpec((_D, 3 * _D), lambda i: (0, 0)),
            pl.BlockSpec((_D, _D), lambda i: (0, 0)),
        ],
        out_specs=pl.BlockSpec((_RB, _D), lambda i: (i, 0)),
        scratch_shapes=[pltpu.VMEM((_RB, _D), jnp.float32)],
        compiler_params=pltpu.CompilerParams(
            dimension_semantics=("parallel",),
            vmem_limit_bytes=64 * 1024 * 1024,
        ),
    )(x2d, w_qkv, w_proj)
    return y2d.reshape(_B, _T, _D)
